# Initial kernel scaffold; baseline (speedup 1.0000x reference)
import jax
import jax.numpy as jnp
from jax.experimental import pallas as pl


def kernel(pred, target):
    raise NotImplementedError("write your pallas kernel here")


# speedup vs baseline: 1.8142x; 1.8142x over previous
import jax
import jax.numpy as jnp
from jax.experimental import pallas as pl
from jax.experimental.pallas import tpu as pltpu

_B, _N, _M = 8, 2048, 2048


def _chamfer_body(p_ref, tT_ref, out_ref):
    b = pl.program_id(0)
    p = p_ref[0]
    tT = tT_ref[0]
    d2 = (p[:, 0:1] - tT[0:1, :]) ** 2
    d2 += (p[:, 1:2] - tT[1:2, :]) ** 2
    d2 += (p[:, 2:3] - tT[2:3, :]) ** 2
    s1 = jnp.sum(jnp.min(d2, axis=1))
    s2 = jnp.sum(jnp.min(d2, axis=0))

    @pl.when(b == 0)
    def _():
        out_ref[0, 0] = 0.0

    out_ref[0, 0] += (s1 + s2) * (1.0 / (_B * _N))


def kernel(pred, target):
    pred = pred.astype(jnp.float32)
    tT = target.astype(jnp.float32).swapaxes(1, 2)
    out = pl.pallas_call(
        _chamfer_body,
        grid=(_B,),
        in_specs=[
            pl.BlockSpec((1, _N, 3), lambda b: (b, 0, 0)),
            pl.BlockSpec((1, 3, _M), lambda b: (b, 0, 0)),
        ],
        out_specs=pl.BlockSpec(memory_space=pltpu.SMEM),
        out_shape=jax.ShapeDtypeStruct((1, 1), jnp.float32),
        compiler_params=pltpu.CompilerParams(
            dimension_semantics=("arbitrary",),
        ),
    )(pred, tT)
    return out[0, 0]
